# baseline (speedup 1.0000x reference)
import functools

import jax
import jax.numpy as jnp
from jax import lax
from jax.experimental import pallas as pl
from jax.experimental.pallas import tpu as pltpu
from jax.experimental.pallas import tpu_sc as plsc

N0 = 10000
N1 = 2500
NC = 2
NS = 16
NT = NC * NS

CH0, NCH0 = 120, 16
E0P = NT * NCH0 * CH0
CH1, NCH1 = 96, 5
E1P = NT * NCH1 * CH1
R0 = 10112
SLAB0 = R0 // NS
R1 = 2560
SLAB1 = R1 // NS


def _pad_edges(ei, epad, nch, ch, slop_row):
  e = ei.shape[1]
  src = jnp.concatenate([ei[0], jnp.zeros((epad - e,), jnp.int32)])
  dst = jnp.concatenate([ei[1], jnp.full((epad - e,), slop_row, jnp.int32)])
  return src.reshape(NT, nch, ch), dst.reshape(NT, nch, ch)


NB = 8


def _sc_seg_feat2(x1, x2, src3, dst3):
  mesh = plsc.VectorSubcoreMesh(core_axis_name="c", subcore_axis_name="s")

  @functools.partial(
      pl.kernel,
      out_type=(jax.ShapeDtypeStruct((NC, R0, 32), jnp.float32),
                jax.ShapeDtypeStruct((NC, R0, 32), jnp.float32)),
      mesh=mesh,
      scratch_types=[
          pltpu.VMEM((NCH0, CH0), jnp.int32),
          pltpu.VMEM((NCH0, CH0), jnp.int32),
          pltpu.VMEM((NB, CH0, 32), jnp.float32),
          pltpu.VMEM((NB, CH0, 32), jnp.float32),
          pltpu.VMEM((SLAB0, 32), jnp.float32),
          pltpu.VMEM_SHARED((R0, 32), jnp.float32),
          pltpu.VMEM_SHARED((R0, 32), jnp.float32),
          pltpu.SemaphoreType.DMA,
          pltpu.SemaphoreType.DMA,
      ],
      compiler_params=pltpu.CompilerParams(needs_layout_passes=False,
                                           use_tc_tiling_on_sc=False),
  )
  def k(x1_hbm, x2_hbm, src_hbm, dst_hbm, s1_hbm, s2_hbm,
        srcv, dstv, rows1, rows2, zf, acc1, acc2, gsem, ssem):
    c = lax.axis_index("c")
    s = lax.axis_index("s")
    t = c * NS + s
    zv = jnp.zeros((16,), jnp.float32)

    def fill_zero(i, _):
      for r in range(8):
        for f in range(2):
          zf[8 * i + r, pl.ds(16 * f, 16)] = zv
      return 0
    lax.fori_loop(0, SLAB0 // 8, fill_zero, 0)

    pltpu.sync_copy(zf, acc1.at[pl.ds(s * SLAB0, SLAB0)])
    pltpu.sync_copy(zf, acc2.at[pl.ds(s * SLAB0, SLAB0)])
    pltpu.sync_copy(src_hbm.at[t], srcv)
    pltpu.sync_copy(dst_hbm.at[t], dstv)
    plsc.subcore_barrier()

    def round_(r, _):
      base = r * NB
      gd = []
      for b in range(NB):
        gd.append(pltpu.async_copy(x1_hbm.at[srcv.at[base + b]], rows1.at[b],
                                   gsem))
        gd.append(pltpu.async_copy(x2_hbm.at[srcv.at[base + b]], rows2.at[b],
                                   gsem))
      sd = []
      for b in range(NB):
        gd[2 * b].wait()
        gd[2 * b + 1].wait()
        sd.append(pltpu.async_copy(rows1.at[b], acc1.at[dstv.at[base + b]],
                                   ssem, add=True))
        sd.append(pltpu.async_copy(rows2.at[b], acc2.at[dstv.at[base + b]],
                                   ssem, add=True))
      for d in sd:
        d.wait()
      return 0
    lax.fori_loop(0, NCH0 // NB, round_, 0)

    plsc.subcore_barrier()
    sl = pl.ds(s * SLAB0, SLAB0)
    pltpu.sync_copy(acc1.at[sl], s1_hbm.at[c, sl])
    pltpu.sync_copy(acc2.at[sl], s2_hbm.at[c, sl])

  return k(x1, x2, src3, dst3)


def _sc_seg_feat(x, src3, dst3, nfeat, with_counts):
  mesh = plsc.VectorSubcoreMesh(core_axis_name="c", subcore_axis_name="s")

  outs = [jax.ShapeDtypeStruct((NC, R0, nfeat), jnp.float32)]
  scratch = [
      pltpu.VMEM((NCH0, CH0), jnp.int32),
      pltpu.VMEM((NCH0, CH0), jnp.int32),
      pltpu.VMEM((NB, CH0, nfeat), jnp.float32),
      pltpu.VMEM((SLAB0, nfeat), jnp.float32),
      pltpu.VMEM_SHARED((R0, nfeat), jnp.float32),
      pltpu.SemaphoreType.DMA,
      pltpu.SemaphoreType.DMA,
  ]
  if with_counts:
    outs.append(jax.ShapeDtypeStruct((NC, R0, 16), jnp.float32))
    scratch += [
        pltpu.VMEM((CH0, 16), jnp.float32),
        pltpu.VMEM((SLAB0, 16), jnp.float32),
        pltpu.VMEM_SHARED((R0, 16), jnp.float32),
    ]

  @functools.partial(
      pl.kernel,
      out_type=tuple(outs) if with_counts else outs[0],
      mesh=mesh,
      scratch_types=scratch,
      compiler_params=pltpu.CompilerParams(needs_layout_passes=False,
                                           use_tc_tiling_on_sc=False),
  )
  def k(x_hbm, src_hbm, dst_hbm, sums_hbm, *rest):
    if with_counts:
      (cnts_hbm, srcv, dstv, rows, zf, accs, gsem, ssem,
       ones, zc, accc) = rest
    else:
      srcv, dstv, rows, zf, accs, gsem, ssem = rest
    c = lax.axis_index("c")
    s = lax.axis_index("s")
    t = c * NS + s
    zv = jnp.zeros((16,), jnp.float32)
    ov = jnp.ones((16,), jnp.float32)

    def fill_zero(i, _):
      for r in range(8):
        for f in range(nfeat // 16):
          zf[8 * i + r, pl.ds(16 * f, 16)] = zv
        if with_counts:
          zc[8 * i + r, pl.ds(0, 16)] = zv
      return 0
    lax.fori_loop(0, SLAB0 // 8, fill_zero, 0)
    if with_counts:
      def fill_ones(i, _):
        for r in range(8):
          ones[8 * i + r, pl.ds(0, 16)] = ov
        return 0
      lax.fori_loop(0, CH0 // 8, fill_ones, 0)

    pltpu.sync_copy(zf, accs.at[pl.ds(s * SLAB0, SLAB0)])
    if with_counts:
      pltpu.sync_copy(zc, accc.at[pl.ds(s * SLAB0, SLAB0)])
    pltpu.sync_copy(src_hbm.at[t], srcv)
    pltpu.sync_copy(dst_hbm.at[t], dstv)
    plsc.subcore_barrier()

    def round_(r, _):
      base = r * NB
      gd = [pltpu.async_copy(x_hbm.at[srcv.at[base + b]], rows.at[b], gsem)
            for b in range(NB)]
      sd = []
      for b in range(NB):
        gd[b].wait()
        sd.append(pltpu.async_copy(rows.at[b], accs.at[dstv.at[base + b]],
                                   ssem, add=True))
        if with_counts:
          sd.append(pltpu.async_copy(ones, accc.at[dstv.at[base + b]],
                                     ssem, add=True))
      for d in sd:
        d.wait()
      return 0
    lax.fori_loop(0, NCH0 // NB, round_, 0)

    plsc.subcore_barrier()
    sl = pl.ds(s * SLAB0, SLAB0)
    pltpu.sync_copy(accs.at[sl], sums_hbm.at[c, sl])
    if with_counts:
      pltpu.sync_copy(accc.at[sl], cnts_hbm.at[c, sl])

  return k(x, src3, dst3)


def _sc_seg_scalar(r1, src3, dst3):
  mesh = plsc.VectorSubcoreMesh(core_axis_name="c", subcore_axis_name="s")

  @functools.partial(
      pl.kernel,
      out_type=(jax.ShapeDtypeStruct((NC * R1,), jnp.float32),
                jax.ShapeDtypeStruct((NC * R1,), jnp.float32)),
      mesh=mesh,
      scratch_types=[
          pltpu.VMEM((N1,), jnp.float32),
          pltpu.VMEM((CH1,), jnp.int32),
          pltpu.VMEM((NCH1, CH1), jnp.int32),
          pltpu.VMEM((CH1,), jnp.float32),
          pltpu.VMEM((CH1,), jnp.float32),
          pltpu.VMEM((SLAB1,), jnp.float32),
          pltpu.VMEM_SHARED((R1,), jnp.float32),
          pltpu.VMEM_SHARED((R1,), jnp.float32),
      ],
      compiler_params=pltpu.CompilerParams(needs_layout_passes=False, use_tc_tiling_on_sc=False),
  )
  def k(r1_hbm, src_hbm, dst_hbm, sums_hbm, cnts_hbm,
        r1v, srcv, dstv, vals, ones, zb, accs, accc):
    c = lax.axis_index("c")
    s = lax.axis_index("s")
    t = c * NS + s
    zv = jnp.zeros((16,), jnp.float32)
    ov = jnp.ones((16,), jnp.float32)

    for i in range(SLAB1 // 16):
      zb[pl.ds(16 * i, 16)] = zv
    for i in range(CH1 // 16):
      ones[pl.ds(16 * i, 16)] = ov

    pltpu.sync_copy(zb, accs.at[pl.ds(s * SLAB1, SLAB1)])
    pltpu.sync_copy(zb, accc.at[pl.ds(s * SLAB1, SLAB1)])
    pltpu.sync_copy(r1_hbm, r1v)
    pltpu.sync_copy(dst_hbm.at[t], dstv)
    plsc.subcore_barrier()

    def chunk(ch, _):
      pltpu.sync_copy(src_hbm.at[t, ch], srcv)
      for j in range(CH1 // 16):
        sidx = srcv[pl.ds(16 * j, 16)]
        vals[pl.ds(16 * j, 16)] = plsc.load_gather(r1v, [sidx])
      pltpu.sync_copy(vals, accs.at[dstv.at[ch]], add=True)
      pltpu.sync_copy(ones, accc.at[dstv.at[ch]], add=True)
      return 0
    lax.fori_loop(0, NCH1, chunk, 0)

    plsc.subcore_barrier()
    sl = pl.ds(s * SLAB1, SLAB1)
    osl = pl.ds(c * R1 + s * SLAB1, SLAB1)
    pltpu.sync_copy(accs.at[sl], zb)
    pltpu.sync_copy(zb, sums_hbm.at[osl])
    pltpu.sync_copy(accc.at[sl], zb)
    pltpu.sync_copy(zb, cnts_hbm.at[osl])

  sums, cnts = k(r1, src3, dst3)
  return sums.reshape(NC, R1), cnts.reshape(NC, R1)


def _tc_prep(z, lin_W, lin_b, U1):
  bs = z.shape[0]
  d = lin_W.shape[0]

  def body(z_ref, w_ref, b_ref, u_ref, o_x, o_r1):
    x = lax.dot_general(
        z_ref[...], w_ref[...], (((1,), (1,)), ((), ())),
        preferred_element_type=jnp.float32) + b_ref[...][None, :]
    o_x[...] = x
    o_r1[...] = jnp.sum(u_ref[...], axis=1, keepdims=True)

  return pl.pallas_call(
      body,
      out_shape=(jax.ShapeDtypeStruct((bs, d), jnp.float32),
                 jax.ShapeDtypeStruct((N1, 1), jnp.float32)))(
          z, lin_W, lin_b, U1)


def _tc_h1(x, m1s, m1c, g0_W, g0_b, bng_g, bng_b, bs):
  total = float(bs * N1)
  nconst = float((bs - 1) * N1)

  def body(x_ref, m_ref, c_ref, w_ref, b_ref, g_ref, bt_ref, o_ref):
    v = x_ref[0:1, 0:16]
    p = jnp.dot(v, w_ref[...], preferred_element_type=jnp.float32)
    ssum = (m_ref[0] + m_ref[1])[:N1][:, None]
    cnt = (c_ref[0] + c_ref[1])[:N1][:, None]
    mbar = ssum / jnp.maximum(cnt, 1.0)
    b = b_ref[...][None, :]
    h = mbar * p + b
    mu = (jnp.sum(h, axis=0, keepdims=True) + nconst * b) / total
    e2 = (jnp.sum(h * h, axis=0, keepdims=True) + nconst * b * b) / total
    var = e2 - mu * mu
    y = (h - mu) / jnp.sqrt(var + 1e-5) * g_ref[...][None, :] + bt_ref[...][None, :]
    o_ref[...] = jnp.where(y >= 0, y, 0.01 * y)

  return pl.pallas_call(
      body, out_shape=jax.ShapeDtypeStruct((N1, 32), jnp.float32))(
          x, m1s, m1c, g0_W, g0_b, bng_g, bng_b)


N1P = 2560


def _tc_u0mm(U0b, h1n):
  def body(u_ref, h_ref, o_ref):
    o_ref[...] = jnp.dot(u_ref[...], h_ref[...].astype(jnp.bfloat16),
                         preferred_element_type=jnp.float32)

  return pl.pallas_call(
      body,
      grid=(10,),
      in_specs=[pl.BlockSpec((N0 // 10, N1P), lambda i: (i, 0)),
                pl.BlockSpec((N1P, 32), lambda i: (0, 0))],
      out_specs=pl.BlockSpec((N0 // 10, 32), lambda i: (i, 0)),
      out_shape=jax.ShapeDtypeStruct((N0, 32), jnp.float32))(U0b, h1n)


def _tc_loc(loc_W, x):
  blk = 16 * N0 // 10

  def body(a_ref, x_ref, o_ref):
    w8 = jnp.broadcast_to(x_ref[0:1, 16:], (8, 128))
    o_ref[...] = lax.dot_general(
        a_ref[...], w8, (((1,), (1,)), ((), ())),
        preferred_element_type=jnp.float32)

  return pl.pallas_call(
      body,
      grid=(10,),
      in_specs=[pl.BlockSpec((blk, 128), lambda i: (i, 0)),
                pl.BlockSpec(x.shape, lambda i: (0, 0))],
      out_specs=pl.BlockSpec((blk, 8), lambda i: (i, 0)),
      out_shape=jax.ShapeDtypeStruct((16 * N0, 8), jnp.float32))(
          loc_W, x)


def _tc_hl(sums, cnts, l0_W, l0_b, bnl_g, bnl_b, bs):
  total = float(bs * N0)
  nconst = float((bs - 1) * N0)

  def body(s_ref, c_ref, w_ref, b_ref, g_ref, bt_ref, o_ref):
    st = s_ref[0] + s_ref[1]
    ct = c_ref[0] + c_ref[1]
    cnt = ct[:N0, 0:1]
    agg = st[:N0, :] / jnp.maximum(cnt, 1.0)
    b = b_ref[...][None, :]
    h = jnp.dot(agg, w_ref[...], preferred_element_type=jnp.float32) + b
    mu = (jnp.sum(h, axis=0, keepdims=True) + nconst * b) / total
    e2 = (jnp.sum(h * h, axis=0, keepdims=True) + nconst * b * b) / total
    var = e2 - mu * mu
    y = (h - mu) / jnp.sqrt(var + 1e-5) * g_ref[...][None, :] + bt_ref[...][None, :]
    o_ref[...] = jnp.where(y >= 0, y, 0.01 * y)

  return pl.pallas_call(
      body, out_shape=jax.ShapeDtypeStruct((N0, 32), jnp.float32))(
          sums, cnts, l0_W, l0_b, bnl_g, bnl_b)


def _tc_final(sums_g, sums_l1, cnts, g1_W, g1_b, l1_W, l1_b, bs):
  def body(sg_ref, sl_ref, c_ref, wg_ref, bg_ref, wl_ref, bl_ref, o_ref):
    i = pl.program_id(0)
    crow = (0.01 * bg_ref[...] + 0.99 * bl_ref[...])[None, :]

    @pl.when(i == 0)
    def _():
      ct = c_ref[0] + c_ref[1]
      cnt = jnp.maximum(ct[:N0, 0:1], 1.0)
      agg_g = (sg_ref[0] + sg_ref[1])[:N0, :] / cnt
      agg_l = (sl_ref[0] + sl_ref[1])[:N0, :] / cnt
      xg = jnp.dot(agg_g, wg_ref[...],
                   preferred_element_type=jnp.float32) + bg_ref[...][None, :]
      xl = jnp.dot(agg_l, wl_ref[...],
                   preferred_element_type=jnp.float32) + bl_ref[...][None, :]
      o_ref[...] = 0.01 * xg + 0.99 * xl

    @pl.when(i != 0)
    def _():
      o_ref[...] = jnp.broadcast_to(crow, (N0, 64))

  full = pl.BlockSpec((NC, R0, 32), lambda i: (0, 0, 0))
  return pl.pallas_call(
      body,
      grid=(bs,),
      in_specs=[full, full,
                pl.BlockSpec((NC, R0, 16), lambda i: (0, 0, 0)),
                pl.BlockSpec((32, 64), lambda i: (0, 0)),
                pl.BlockSpec((64,), lambda i: (0,)),
                pl.BlockSpec((32, 64), lambda i: (0, 0)),
                pl.BlockSpec((64,), lambda i: (0,))],
      out_specs=pl.BlockSpec((N0, 64), lambda i: (i, 0)),
      out_shape=jax.ShapeDtypeStruct((bs * N0, 64), jnp.float32))(
          sums_g, sums_l1, cnts, g1_W, g1_b, l1_W, l1_b)


def kernel(z, batch_size, A0, A1, U0, U1, lin_W, lin_b, loc_W, loc_b,
           g0_W, g0_u, g0_c, g0_b, g1_W, g1_u, g1_c, g1_b,
           l0_W, l0_u, l0_c, l0_b, l1_W, l1_u, l1_c, l1_b,
           bng_g, bng_b, bnl_g, bnl_b):
  del batch_size, g0_u, g0_c, g1_u, g1_c, l0_u, l0_c, l1_u, l1_c
  bs = z.shape[0]

  src0, dst0 = _pad_edges(A0, E0P, NCH0, CH0, N0)
  src1, dst1 = _pad_edges(A1, E1P, NCH1, CH1, N1)

  x, r1 = _tc_prep(z, lin_W, lin_b, U1)

  xl0 = _tc_loc(loc_W, x)[:, 0].reshape(N0, 16) + loc_b.reshape(N0, 16)

  m1s, m1c = _sc_seg_scalar(r1[:, 0], src1, dst1)
  h1n = _tc_h1(x, m1s, m1c, g0_W, g0_b, bng_g, bng_b, bs)
  u0b = jnp.pad(U0.astype(jnp.bfloat16), ((0, 0), (0, N1P - N1)))
  h1np = jnp.pad(h1n, ((0, N1P - N1), (0, 0)))
  y0 = _tc_u0mm(u0b, h1np)

  sums_l0, cnts0 = _sc_seg_feat(xl0, src0, dst0, 16, True)
  hln = _tc_hl(sums_l0, cnts0, l0_W, l0_b, bnl_g, bnl_b, bs)
  sums_l1, sums_g = _sc_seg_feat2(hln, y0, src0, dst0)

  return _tc_final(sums_g, sums_l1, cnts0, g1_W, g1_b, l1_W, l1_b, bs)


# speedup vs baseline: 1.2445x; 1.2445x over previous
import functools

import jax
import jax.numpy as jnp
from jax import lax
from jax.experimental import pallas as pl
from jax.experimental.pallas import tpu as pltpu
from jax.experimental.pallas import tpu_sc as plsc

N0 = 10000
N1 = 2500
NC = 2
NS = 16
NT = NC * NS

CH0, NCH0 = 120, 16
E0P = NT * NCH0 * CH0
CH1, NCH1 = 96, 5
E1P = NT * NCH1 * CH1
R0 = 10112
SLAB0 = R0 // NS
R1 = 2560
SLAB1 = R1 // NS


def _pad_edges(ei, epad, nch, ch, slop_row):
  e = ei.shape[1]
  src = jnp.concatenate([ei[0], jnp.zeros((epad - e,), jnp.int32)])
  dst = jnp.concatenate([ei[1], jnp.full((epad - e,), slop_row, jnp.int32)])
  return src.reshape(NT, nch, ch), dst.reshape(NT, nch, ch)


NB = 8


def _sc_seg_feat2(x1, x2, src3, dst3):
  mesh = plsc.VectorSubcoreMesh(core_axis_name="c", subcore_axis_name="s")

  @functools.partial(
      pl.kernel,
      out_type=(jax.ShapeDtypeStruct((NC, R0, 32), jnp.float32),
                jax.ShapeDtypeStruct((NC, R0, 32), jnp.float32)),
      mesh=mesh,
      scratch_types=[
          pltpu.VMEM((NCH0, CH0), jnp.int32),
          pltpu.VMEM((NCH0, CH0), jnp.int32),
          pltpu.VMEM((NB, CH0, 32), jnp.float32),
          pltpu.VMEM((NB, CH0, 32), jnp.float32),
          pltpu.VMEM((SLAB0, 32), jnp.float32),
          pltpu.VMEM_SHARED((R0, 32), jnp.float32),
          pltpu.VMEM_SHARED((R0, 32), jnp.float32),
          pltpu.SemaphoreType.DMA,
          pltpu.SemaphoreType.DMA,
      ],
      compiler_params=pltpu.CompilerParams(needs_layout_passes=False,
                                           use_tc_tiling_on_sc=False),
  )
  def k(x1_hbm, x2_hbm, src_hbm, dst_hbm, s1_hbm, s2_hbm,
        srcv, dstv, rows1, rows2, zf, acc1, acc2, gsem, ssem):
    c = lax.axis_index("c")
    s = lax.axis_index("s")
    t = c * NS + s
    zv = jnp.zeros((16,), jnp.float32)

    def fill_zero(i, _):
      for r in range(8):
        for f in range(2):
          zf[8 * i + r, pl.ds(16 * f, 16)] = zv
      return 0
    lax.fori_loop(0, SLAB0 // 8, fill_zero, 0)

    pltpu.sync_copy(zf, acc1.at[pl.ds(s * SLAB0, SLAB0)])
    pltpu.sync_copy(zf, acc2.at[pl.ds(s * SLAB0, SLAB0)])
    pltpu.sync_copy(src_hbm.at[t], srcv)
    pltpu.sync_copy(dst_hbm.at[t], dstv)
    plsc.subcore_barrier()

    def round_(r, _):
      base = r * NB
      gd = []
      for b in range(NB):
        gd.append(pltpu.async_copy(x1_hbm.at[srcv.at[base + b]], rows1.at[b],
                                   gsem))
        gd.append(pltpu.async_copy(x2_hbm.at[srcv.at[base + b]], rows2.at[b],
                                   gsem))
      sd = []
      for b in range(NB):
        gd[2 * b].wait()
        gd[2 * b + 1].wait()
        sd.append(pltpu.async_copy(rows1.at[b], acc1.at[dstv.at[base + b]],
                                   ssem, add=True))
        sd.append(pltpu.async_copy(rows2.at[b], acc2.at[dstv.at[base + b]],
                                   ssem, add=True))
      for d in sd:
        d.wait()
      return 0
    lax.fori_loop(0, NCH0 // NB, round_, 0)

    plsc.subcore_barrier()
    sl = pl.ds(s * SLAB0, SLAB0)
    pltpu.sync_copy(acc1.at[sl], s1_hbm.at[c, sl])
    pltpu.sync_copy(acc2.at[sl], s2_hbm.at[c, sl])

  return k(x1, x2, src3, dst3)


def _sc_seg_feat(x, src3, dst3, nfeat, with_counts):
  mesh = plsc.VectorSubcoreMesh(core_axis_name="c", subcore_axis_name="s")

  outs = [jax.ShapeDtypeStruct((NC, R0, nfeat), jnp.float32)]
  scratch = [
      pltpu.VMEM((NCH0, CH0), jnp.int32),
      pltpu.VMEM((NCH0, CH0), jnp.int32),
      pltpu.VMEM((NB, CH0, nfeat), jnp.float32),
      pltpu.VMEM((SLAB0, nfeat), jnp.float32),
      pltpu.VMEM_SHARED((R0, nfeat), jnp.float32),
      pltpu.SemaphoreType.DMA,
      pltpu.SemaphoreType.DMA,
  ]
  if with_counts:
    outs.append(jax.ShapeDtypeStruct((NC, R0, 16), jnp.float32))
    scratch += [
        pltpu.VMEM((CH0, 16), jnp.float32),
        pltpu.VMEM((SLAB0, 16), jnp.float32),
        pltpu.VMEM_SHARED((R0, 16), jnp.float32),
    ]

  @functools.partial(
      pl.kernel,
      out_type=tuple(outs) if with_counts else outs[0],
      mesh=mesh,
      scratch_types=scratch,
      compiler_params=pltpu.CompilerParams(needs_layout_passes=False,
                                           use_tc_tiling_on_sc=False),
  )
  def k(x_hbm, src_hbm, dst_hbm, sums_hbm, *rest):
    if with_counts:
      (cnts_hbm, srcv, dstv, rows, zf, accs, gsem, ssem,
       ones, zc, accc) = rest
    else:
      srcv, dstv, rows, zf, accs, gsem, ssem = rest
    c = lax.axis_index("c")
    s = lax.axis_index("s")
    t = c * NS + s
    zv = jnp.zeros((16,), jnp.float32)
    ov = jnp.ones((16,), jnp.float32)

    def fill_zero(i, _):
      for r in range(8):
        for f in range(nfeat // 16):
          zf[8 * i + r, pl.ds(16 * f, 16)] = zv
        if with_counts:
          zc[8 * i + r, pl.ds(0, 16)] = zv
      return 0
    lax.fori_loop(0, SLAB0 // 8, fill_zero, 0)
    if with_counts:
      def fill_ones(i, _):
        for r in range(8):
          ones[8 * i + r, pl.ds(0, 16)] = ov
        return 0
      lax.fori_loop(0, CH0 // 8, fill_ones, 0)

    pltpu.sync_copy(zf, accs.at[pl.ds(s * SLAB0, SLAB0)])
    if with_counts:
      pltpu.sync_copy(zc, accc.at[pl.ds(s * SLAB0, SLAB0)])
    pltpu.sync_copy(src_hbm.at[t], srcv)
    pltpu.sync_copy(dst_hbm.at[t], dstv)
    plsc.subcore_barrier()

    def round_(r, _):
      base = r * NB
      gd = [pltpu.async_copy(x_hbm.at[srcv.at[base + b]], rows.at[b], gsem)
            for b in range(NB)]
      sd = []
      for b in range(NB):
        gd[b].wait()
        sd.append(pltpu.async_copy(rows.at[b], accs.at[dstv.at[base + b]],
                                   ssem, add=True))
        if with_counts:
          sd.append(pltpu.async_copy(ones, accc.at[dstv.at[base + b]],
                                     ssem, add=True))
      for d in sd:
        d.wait()
      return 0
    lax.fori_loop(0, NCH0 // NB, round_, 0)

    plsc.subcore_barrier()
    sl = pl.ds(s * SLAB0, SLAB0)
    pltpu.sync_copy(accs.at[sl], sums_hbm.at[c, sl])
    if with_counts:
      pltpu.sync_copy(accc.at[sl], cnts_hbm.at[c, sl])

  return k(x, src3, dst3)


def _sc_seg_scalar(r1, src3, dst3):
  mesh = plsc.VectorSubcoreMesh(core_axis_name="c", subcore_axis_name="s")

  @functools.partial(
      pl.kernel,
      out_type=(jax.ShapeDtypeStruct((NC * R1,), jnp.float32),
                jax.ShapeDtypeStruct((NC * R1,), jnp.float32)),
      mesh=mesh,
      scratch_types=[
          pltpu.VMEM((N1,), jnp.float32),
          pltpu.VMEM((CH1,), jnp.int32),
          pltpu.VMEM((NCH1, CH1), jnp.int32),
          pltpu.VMEM((CH1,), jnp.float32),
          pltpu.VMEM((CH1,), jnp.float32),
          pltpu.VMEM((SLAB1,), jnp.float32),
          pltpu.VMEM_SHARED((R1,), jnp.float32),
          pltpu.VMEM_SHARED((R1,), jnp.float32),
      ],
      compiler_params=pltpu.CompilerParams(needs_layout_passes=False, use_tc_tiling_on_sc=False),
  )
  def k(r1_hbm, src_hbm, dst_hbm, sums_hbm, cnts_hbm,
        r1v, srcv, dstv, vals, ones, zb, accs, accc):
    c = lax.axis_index("c")
    s = lax.axis_index("s")
    t = c * NS + s
    zv = jnp.zeros((16,), jnp.float32)
    ov = jnp.ones((16,), jnp.float32)

    for i in range(SLAB1 // 16):
      zb[pl.ds(16 * i, 16)] = zv
    for i in range(CH1 // 16):
      ones[pl.ds(16 * i, 16)] = ov

    pltpu.sync_copy(zb, accs.at[pl.ds(s * SLAB1, SLAB1)])
    pltpu.sync_copy(zb, accc.at[pl.ds(s * SLAB1, SLAB1)])
    pltpu.sync_copy(r1_hbm, r1v)
    pltpu.sync_copy(dst_hbm.at[t], dstv)
    plsc.subcore_barrier()

    def chunk(ch, _):
      pltpu.sync_copy(src_hbm.at[t, ch], srcv)
      for j in range(CH1 // 16):
        sidx = srcv[pl.ds(16 * j, 16)]
        vals[pl.ds(16 * j, 16)] = plsc.load_gather(r1v, [sidx])
      pltpu.sync_copy(vals, accs.at[dstv.at[ch]], add=True)
      pltpu.sync_copy(ones, accc.at[dstv.at[ch]], add=True)
      return 0
    lax.fori_loop(0, NCH1, chunk, 0)

    plsc.subcore_barrier()
    sl = pl.ds(s * SLAB1, SLAB1)
    osl = pl.ds(c * R1 + s * SLAB1, SLAB1)
    pltpu.sync_copy(accs.at[sl], zb)
    pltpu.sync_copy(zb, sums_hbm.at[osl])
    pltpu.sync_copy(accc.at[sl], zb)
    pltpu.sync_copy(zb, cnts_hbm.at[osl])

  sums, cnts = k(r1, src3, dst3)
  return sums.reshape(NC, R1), cnts.reshape(NC, R1)


def _tc_prep(z, lin_W, lin_b, U1):
  bs = z.shape[0]
  d = lin_W.shape[0]

  def body(z_ref, w_ref, b_ref, u_ref, o_x, o_r1):
    x = lax.dot_general(
        z_ref[...], w_ref[...], (((1,), (1,)), ((), ())),
        preferred_element_type=jnp.float32) + b_ref[...][None, :]
    o_x[...] = x
    o_r1[...] = jnp.sum(u_ref[...], axis=1, keepdims=True)

  return pl.pallas_call(
      body,
      out_shape=(jax.ShapeDtypeStruct((bs, d), jnp.float32),
                 jax.ShapeDtypeStruct((N1, 1), jnp.float32)))(
          z, lin_W, lin_b, U1)


def _tc_h1(x, m1s, m1c, g0_W, g0_b, bng_g, bng_b, bs):
  total = float(bs * N1)
  nconst = float((bs - 1) * N1)

  def body(x_ref, m_ref, c_ref, w_ref, b_ref, g_ref, bt_ref, o_ref):
    v = x_ref[0:1, 0:16]
    p = jnp.dot(v, w_ref[...], preferred_element_type=jnp.float32)
    ssum = (m_ref[0] + m_ref[1])[:N1][:, None]
    cnt = (c_ref[0] + c_ref[1])[:N1][:, None]
    mbar = ssum / jnp.maximum(cnt, 1.0)
    b = b_ref[...][None, :]
    h = mbar * p + b
    mu = (jnp.sum(h, axis=0, keepdims=True) + nconst * b) / total
    e2 = (jnp.sum(h * h, axis=0, keepdims=True) + nconst * b * b) / total
    var = e2 - mu * mu
    y = (h - mu) / jnp.sqrt(var + 1e-5) * g_ref[...][None, :] + bt_ref[...][None, :]
    o_ref[...] = jnp.where(y >= 0, y, 0.01 * y)

  return pl.pallas_call(
      body, out_shape=jax.ShapeDtypeStruct((N1, 32), jnp.float32))(
          x, m1s, m1c, g0_W, g0_b, bng_g, bng_b)


def _tc_u0mm(U0, h1n):
  blk = N0 // 10

  def body(u_hbm, h_ref, o_ref, ubuf, sem0, sem1):
    i = pl.program_id(0)
    sems = [sem0, sem1]

    def start(j, slot):
      return pltpu.make_async_copy(
          u_hbm.at[pl.ds(j * blk, blk), :], ubuf.at[slot], sems[slot])

    @pl.when(i == 0)
    def _():
      start(0, 0).start()

    @pl.when(i + 1 < 10)
    def _():
      @pl.when(i % 2 == 0)
      def _():
        start(i + 1, 1).start()

      @pl.when(i % 2 == 1)
      def _():
        start(i + 1, 0).start()

    @pl.when(i % 2 == 0)
    def _():
      start(i, 0).wait()
      o_ref[...] = jnp.dot(ubuf[0], h_ref[...],
                           preferred_element_type=jnp.float32)

    @pl.when(i % 2 == 1)
    def _():
      start(i, 1).wait()
      o_ref[...] = jnp.dot(ubuf[1], h_ref[...],
                           preferred_element_type=jnp.float32)

  return pl.pallas_call(
      body,
      grid=(10,),
      in_specs=[pl.BlockSpec(memory_space=pltpu.MemorySpace.HBM),
                pl.BlockSpec((N1, 32), lambda i: (0, 0))],
      out_specs=pl.BlockSpec((blk, 32), lambda i: (i, 0)),
      out_shape=jax.ShapeDtypeStruct((N0, 32), jnp.float32),
      scratch_shapes=[pltpu.VMEM((2, blk, N1), jnp.float32),
                      pltpu.SemaphoreType.DMA,
                      pltpu.SemaphoreType.DMA])(U0, h1n)


def _tc_loc(locW3, x, loc_b2):
  def body(a_ref, x_ref, b_ref, o_ref):
    w = x_ref[0:1, 16:]
    o_ref[...] = jnp.sum(a_ref[...] * w[None, :, :], axis=-1) + b_ref[...]

  return pl.pallas_call(
      body,
      grid=(10,),
      in_specs=[pl.BlockSpec((N0 // 10, 16, 128), lambda i: (i, 0, 0)),
                pl.BlockSpec(x.shape, lambda i: (0, 0)),
                pl.BlockSpec((N0 // 10, 16), lambda i: (i, 0))],
      out_specs=pl.BlockSpec((N0 // 10, 16), lambda i: (i, 0)),
      out_shape=jax.ShapeDtypeStruct((N0, 16), jnp.float32))(
          locW3, x, loc_b2)


def _tc_hl(sums, cnts, l0_W, l0_b, bnl_g, bnl_b, bs):
  total = float(bs * N0)
  nconst = float((bs - 1) * N0)

  def body(s_ref, c_ref, w_ref, b_ref, g_ref, bt_ref, o_ref):
    st = s_ref[0] + s_ref[1]
    ct = c_ref[0] + c_ref[1]
    cnt = ct[:N0, 0:1]
    agg = st[:N0, :] / jnp.maximum(cnt, 1.0)
    b = b_ref[...][None, :]
    h = jnp.dot(agg, w_ref[...], preferred_element_type=jnp.float32) + b
    mu = (jnp.sum(h, axis=0, keepdims=True) + nconst * b) / total
    e2 = (jnp.sum(h * h, axis=0, keepdims=True) + nconst * b * b) / total
    var = e2 - mu * mu
    y = (h - mu) / jnp.sqrt(var + 1e-5) * g_ref[...][None, :] + bt_ref[...][None, :]
    o_ref[...] = jnp.where(y >= 0, y, 0.01 * y)

  return pl.pallas_call(
      body, out_shape=jax.ShapeDtypeStruct((N0, 32), jnp.float32))(
          sums, cnts, l0_W, l0_b, bnl_g, bnl_b)


def _tc_final(sums_g, sums_l1, cnts, g1_W, g1_b, l1_W, l1_b, bs):
  def body(sg_ref, sl_ref, c_ref, wg_ref, bg_ref, wl_ref, bl_ref, o_ref):
    i = pl.program_id(0)
    crow = (0.01 * bg_ref[...] + 0.99 * bl_ref[...])[None, :]

    @pl.when(i == 0)
    def _():
      ct = c_ref[0] + c_ref[1]
      cnt = jnp.maximum(ct[:N0, 0:1], 1.0)
      agg_g = (sg_ref[0] + sg_ref[1])[:N0, :] / cnt
      agg_l = (sl_ref[0] + sl_ref[1])[:N0, :] / cnt
      xg = jnp.dot(agg_g, wg_ref[...],
                   preferred_element_type=jnp.float32) + bg_ref[...][None, :]
      xl = jnp.dot(agg_l, wl_ref[...],
                   preferred_element_type=jnp.float32) + bl_ref[...][None, :]
      o_ref[...] = 0.01 * xg + 0.99 * xl

    @pl.when(i != 0)
    def _():
      o_ref[...] = jnp.broadcast_to(crow, (N0, 64))

  full = pl.BlockSpec((NC, R0, 32), lambda i: (0, 0, 0))
  return pl.pallas_call(
      body,
      grid=(bs,),
      in_specs=[full, full,
                pl.BlockSpec((NC, R0, 16), lambda i: (0, 0, 0)),
                pl.BlockSpec((32, 64), lambda i: (0, 0)),
                pl.BlockSpec((64,), lambda i: (0,)),
                pl.BlockSpec((32, 64), lambda i: (0, 0)),
                pl.BlockSpec((64,), lambda i: (0,))],
      out_specs=pl.BlockSpec((N0, 64), lambda i: (i, 0)),
      out_shape=jax.ShapeDtypeStruct((bs * N0, 64), jnp.float32))(
          sums_g, sums_l1, cnts, g1_W, g1_b, l1_W, l1_b)


def kernel(z, batch_size, A0, A1, U0, U1, lin_W, lin_b, loc_W, loc_b,
           g0_W, g0_u, g0_c, g0_b, g1_W, g1_u, g1_c, g1_b,
           l0_W, l0_u, l0_c, l0_b, l1_W, l1_u, l1_c, l1_b,
           bng_g, bng_b, bnl_g, bnl_b):
  del batch_size, g0_u, g0_c, g1_u, g1_c, l0_u, l0_c, l1_u, l1_c
  bs = z.shape[0]

  src0, dst0 = _pad_edges(A0, E0P, NCH0, CH0, N0)
  src1, dst1 = _pad_edges(A1, E1P, NCH1, CH1, N1)

  x, r1 = _tc_prep(z, lin_W, lin_b, U1)

  xl0 = _tc_loc(loc_W.reshape(N0, 16, 128), x, loc_b.reshape(N0, 16))

  m1s, m1c = _sc_seg_scalar(r1[:, 0], src1, dst1)
  h1n = _tc_h1(x, m1s, m1c, g0_W, g0_b, bng_g, bng_b, bs)
  y0 = _tc_u0mm(U0, h1n)

  sums_l0, cnts0 = _sc_seg_feat(xl0, src0, dst0, 16, True)
  hln = _tc_hl(sums_l0, cnts0, l0_W, l0_b, bnl_g, bnl_b, bs)
  sums_l1, sums_g = _sc_seg_feat2(hln, y0, src0, dst0)

  return _tc_final(sums_g, sums_l1, cnts0, g1_W, g1_b, l1_W, l1_b, bs)
